# baseline (speedup 1.0000x reference)
import functools

import jax
import jax.numpy as jnp
from jax import lax
from jax.experimental import pallas as pl
from jax.experimental.pallas import tpu as pltpu
from jax.experimental.pallas import tpu_sc as plsc

_DIM = 32
_B = 4096 * 50
_NW = 32
_BPW = _B // _NW
_C = 128
_NCH = _BPW // _C

_mesh = plsc.VectorSubcoreMesh(core_axis_name="c", subcore_axis_name="s")


@functools.partial(
    pl.kernel,
    mesh=_mesh,
    out_type=jax.ShapeDtypeStruct((_B, _DIM), jnp.float32),
    scratch_types=[
        pltpu.VMEM((_BPW,), jnp.int32),
        pltpu.VMEM((_C, _DIM), jnp.float32),
        pltpu.SemaphoreType.DMA,
    ],
    compiler_params=pltpu.CompilerParams(use_tc_tiling_on_sc=False),
)
def _emb(table_hbm, idx_hbm, out_hbm, idx_v, rows_v, gsem):
    wid = lax.axis_index("s") * 2 + lax.axis_index("c")
    pltpu.sync_copy(idx_hbm.at[pl.ds(pl.multiple_of(wid * _BPW, _BPW), _BPW)], idx_v)
    out_base = wid * _BPW

    def chunk(j, carry):
        cb = pl.multiple_of(j * _C, _C)
        pltpu.async_copy(table_hbm.at[idx_v.at[pl.ds(cb, _C)]], rows_v, gsem).wait()
        pltpu.sync_copy(
            rows_v, out_hbm.at[pl.ds(pl.multiple_of(out_base + j * _C, _C), _C)]
        )
        return carry

    lax.fori_loop(0, _NCH, chunk, 0)


@jax.jit
def kernel(inputs, e):
    idx = inputs.reshape(_B).astype(jnp.int32)
    out = _emb(e, idx)
    return out.reshape(inputs.shape + (_DIM,))


# speedup vs baseline: 1.0459x; 1.0459x over previous
import functools

import jax
import jax.numpy as jnp
from jax import lax
from jax.experimental import pallas as pl
from jax.experimental.pallas import tpu as pltpu
from jax.experimental.pallas import tpu_sc as plsc

_DIM = 32
_B = 4096 * 50
_NW = 32
_BPW = _B // _NW
_C = 128
_NCH = _BPW // _C

_NBUF = 8
_LOOK = 4

_mesh = plsc.VectorSubcoreMesh(core_axis_name="c", subcore_axis_name="s")


@functools.partial(
    pl.kernel,
    mesh=_mesh,
    out_type=jax.ShapeDtypeStruct((_B, _DIM), jnp.float32),
    scratch_types=[
        pltpu.VMEM((_BPW,), jnp.int32),
        pltpu.VMEM((_NBUF, _C, _DIM), jnp.float32),
        pltpu.SemaphoreType.DMA,
        pltpu.SemaphoreType.DMA,
    ],
    compiler_params=pltpu.CompilerParams(use_tc_tiling_on_sc=False),
)
def _emb(table_hbm, idx_hbm, out_hbm, idx_v, rows_v, gsem, wsem):
    wid = lax.axis_index("s") * 2 + lax.axis_index("c")
    pltpu.sync_copy(idx_hbm.at[pl.ds(pl.multiple_of(wid * _BPW, _BPW), _BPW)], idx_v)
    out_base = wid * _BPW

    def fire_gather(j):
        cb = pl.multiple_of(j * _C, _C)
        pltpu.async_copy(
            table_hbm.at[idx_v.at[pl.ds(cb, _C)]],
            rows_v.at[lax.rem(j, _NBUF)],
            gsem,
        )

    def wait_gather(b):
        pltpu.make_async_copy(
            table_hbm.at[pl.ds(0, _C)], rows_v.at[b], gsem
        ).wait()

    def wait_one_write():
        pltpu.make_async_copy(
            rows_v.at[0], out_hbm.at[pl.ds(0, _C)], wsem
        ).wait()

    for j in range(_LOOK):
        fire_gather(j)

    def step(j, carry):
        b = lax.rem(j, _NBUF)
        wait_gather(b)
        ob = pl.multiple_of(out_base + j * _C, _C)
        pltpu.async_copy(rows_v.at[b], out_hbm.at[pl.ds(ob, _C)], wsem)
        nxt = j + _LOOK

        @pl.when(nxt < _NCH)
        def _fire_next():
            @pl.when(j >= _NBUF - _LOOK)
            def _retire():
                wait_one_write()

            fire_gather(nxt)

        return carry

    lax.fori_loop(0, _NCH, step, 0)

    for _ in range(min(_NBUF, _NCH)):
        wait_one_write()


@jax.jit
def kernel(inputs, e):
    idx = inputs.reshape(_B).astype(jnp.int32)
    out = _emb(e, idx)
    return out.reshape(inputs.shape + (_DIM,))


# speedup vs baseline: 1.3038x; 1.2467x over previous
import functools

import jax
import jax.numpy as jnp
from jax import lax
from jax.experimental import pallas as pl
from jax.experimental.pallas import tpu as pltpu
from jax.experimental.pallas import tpu_sc as plsc

_DIM = 32
_SEQ = 50
_BATCH = 4096
_B = _BATCH * _SEQ
_NW = 32
_C = 128
_NB = 2

_mesh = plsc.VectorSubcoreMesh(core_axis_name="c", subcore_axis_name="s")


@functools.partial(
    pl.kernel,
    mesh=_mesh,
    out_type=jax.ShapeDtypeStruct((_SEQ, _DIM // 8, _BATCH // _C, 8, _C), jnp.float32),
    scratch_types=[
        pltpu.VMEM((_SEQ * _C,), jnp.int32),
        pltpu.VMEM((_SEQ * _C,), jnp.int32),
        pltpu.VMEM((_NB, _C, _DIM), jnp.float32),
        pltpu.VMEM((_NB, _DIM // 8, 8, _C), jnp.float32),
        pltpu.SemaphoreType.DMA,
        pltpu.SemaphoreType.DMA,
    ],
    compiler_params=pltpu.CompilerParams(use_tc_tiling_on_sc=False, needs_layout_passes=False),
)
def _emb(table_hbm, idx_hbm, out_hbm, idx_v, cidx_v, rows_v, trans_v, gsem, wsem):
    wid = lax.axis_index("s") * 2 + lax.axis_index("c")
    npw = _SEQ * _C
    pltpu.sync_copy(idx_hbm.at[pl.ds(pl.multiple_of(wid * npw, npw), npw)], idx_v)

    iota = jnp.arange(16, dtype=jnp.int32)
    iota50 = iota * _SEQ
    iota128 = iota * _C
    chi = lax.shift_right_logical(iota, 3)
    clo = lax.bitwise_and(iota, 7)

    def regroup(t, carry):
        for g in range(8):
            vals = plsc.load_gather(idx_v, [iota50 + (t + 800 * g)])
            cidx_v[pl.ds(t * _C + 16 * g, 16)] = vals
        return carry

    lax.fori_loop(0, _SEQ, regroup, 0)

    def fire_gather(t):
        pltpu.async_copy(
            table_hbm.at[cidx_v.at[pl.ds(t * _C, _C)]],
            rows_v.at[lax.rem(t, _NB)],
            gsem,
        )

    def wait_gather(b):
        pltpu.make_async_copy(
            table_hbm.at[pl.ds(0, _C)], rows_v.at[b], gsem
        ).wait()

    def wait_one_write():
        pltpu.make_async_copy(
            trans_v.at[0], out_hbm.at[0, :, 0], wsem
        ).wait()

    fire_gather(0)

    def step(t, carry):
        b = lax.rem(t, _NB)
        wait_gather(b)

        @pl.when(t + 1 < _SEQ)
        def _fire_next():
            fire_gather(t + 1)

        @pl.when(t >= _NB)
        def _retire():
            wait_one_write()

        rows = rows_v.at[b]
        tr = trans_v.at[b]

        def tpose(bl, c2):
            g0 = rows[bl, pl.ds(0, 16)]
            g1 = rows[bl, pl.ds(16, 16)]
            plsc.store_scatter(tr, [chi, clo, jnp.full((16,), bl, jnp.int32)], g0)
            plsc.store_scatter(tr, [chi + 2, clo, jnp.full((16,), bl, jnp.int32)], g1)
            return c2

        lax.fori_loop(0, _C, tpose, 0)
        pltpu.async_copy(trans_v.at[b], out_hbm.at[t, :, wid], wsem)
        return carry

    lax.fori_loop(0, _SEQ, step, 0)

    for _ in range(_NB):
        wait_one_write()


@jax.jit
def kernel(inputs, e):
    idx = inputs.reshape(_B).astype(jnp.int32)
    out5 = _emb(e, idx)
    return out5.transpose(2, 4, 0, 1, 3).reshape(_BATCH, _SEQ, _DIM)
